# Initial kernel scaffold; baseline (speedup 1.0000x reference)
import jax
import jax.numpy as jnp
from jax import lax
from jax.experimental import pallas as pl
from jax.experimental.pallas import tpu as pltpu
from jax.experimental.pallas import tpu_sc as plsc

B, F = 4096, 26
V, D = 100000, 64
L = 16
NC, NS = 2, 16
NW = NC * NS
ROWS_W = B // NW
NBLK = ROWS_W // L
IDX_BLK = F * L
NCH = 4
CH = IDX_BLK // NCH
DCH = D // L


def _sc_body(idx_hbm, vals_hbm, fo_hbm, so_hbm,
             pooled_hbm, sumsq_hbm, fo_out_hbm,
             idx_v, vals_v, emb_v, fo_v, pooled_v, sumsq_v, foacc_v,
             sem_e, sem_f):
    wid = lax.axis_index("s") * NC + lax.axis_index("c")
    pltpu.sync_copy(idx_hbm.at[pl.ds(wid * (NBLK * NCH), NBLK * NCH)], idx_v)
    pltpu.sync_copy(vals_hbm.at[pl.ds(wid * (NBLK * IDX_BLK), NBLK * IDX_BLK)],
                    vals_v)

    def block_body(blk, carry):
        for c in range(NCH):
            pltpu.async_copy(so_hbm.at[idx_v.at[blk * NCH + c]],
                             emb_v.at[pl.ds(c * CH, CH)], sem_e)
        for c in range(NCH):
            pltpu.async_copy(fo_hbm.at[idx_v.at[blk * NCH + c]],
                             fo_v.at[pl.ds(c * CH, CH)], sem_f)
        for c in range(NCH):
            pltpu.make_async_copy(so_hbm.at[idx_v.at[blk * NCH + c]],
                                  emb_v.at[pl.ds(c * CH, CH)], sem_e).wait()
        for c in range(NCH):
            pltpu.make_async_copy(fo_hbm.at[idx_v.at[blk * NCH + c]],
                                  fo_v.at[pl.ds(c * CH, CH)], sem_f).wait()

        vbase = blk * IDX_BLK
        acc = jnp.zeros((L,), jnp.float32)
        for f in range(F):
            acc = acc + fo_v[pl.ds(f * L, L)] * vals_v[pl.ds(vbase + f * L, L)]
        foacc_v[pl.ds(blk * L, L)] = acc

        def row_body(r, rc):
            row = blk * L + r
            accs = [jnp.zeros((L,), jnp.float32) for _ in range(2 * DCH)]
            for f in range(F):
                vidx = jnp.full((L,), vbase + f * L + r, jnp.int32)
                vv = plsc.load_gather(vals_v, [vidx])
                for c in range(DCH):
                    t = emb_v[f * L + r, pl.ds(c * L, L)] * vv
                    accs[c] = accs[c] + t
                    accs[DCH + c] = accs[DCH + c] + t * t
            for c in range(DCH):
                pooled_v[row, pl.ds(c * L, L)] = accs[c]
                sumsq_v[row, pl.ds(c * L, L)] = accs[DCH + c]
            return rc

        lax.fori_loop(0, L, row_body, 0)
        return carry

    lax.fori_loop(0, NBLK, block_body, 0)

    pltpu.sync_copy(pooled_v, pooled_hbm.at[pl.ds(wid * ROWS_W, ROWS_W)])
    pltpu.sync_copy(sumsq_v, sumsq_hbm.at[pl.ds(wid * ROWS_W, ROWS_W)])
    pltpu.sync_copy(foacc_v, fo_out_hbm.at[pl.ds(wid * ROWS_W, ROWS_W)])


_sc_call = pl.kernel(
    _sc_body,
    out_type=(
        jax.ShapeDtypeStruct((B, D), jnp.float32),
        jax.ShapeDtypeStruct((B, D), jnp.float32),
        jax.ShapeDtypeStruct((B,), jnp.float32),
    ),
    mesh=plsc.VectorSubcoreMesh(core_axis_name="c", subcore_axis_name="s"),
    scratch_types=(
        pltpu.VMEM((NBLK * NCH, CH), jnp.int32),
        pltpu.VMEM((NBLK * IDX_BLK,), jnp.float32),
        pltpu.VMEM((IDX_BLK, D), jnp.float32),
        pltpu.VMEM((IDX_BLK,), jnp.float32),
        pltpu.VMEM((ROWS_W, D), jnp.float32),
        pltpu.VMEM((ROWS_W, D), jnp.float32),
        pltpu.VMEM((ROWS_W,), jnp.float32),
        pltpu.SemaphoreType.DMA,
        pltpu.SemaphoreType.DMA,
    ),
)


def _tc_body(pooled_ref, sumsq_ref, fo_ref,
             W0_ref, b0_ref, g0_ref, be0_ref,
             W1_ref, b1_ref, g1_ref, be1_ref,
             W2_ref, b2_ref, g2_ref, be2_ref,
             Wo_ref, bo_ref, out_ref):
    p = pooled_ref[:]
    second = 0.5 * jnp.sum(p * p - sumsq_ref[:], axis=1)
    x = p
    for W_ref, b_ref, g_ref, be_ref in (
            (W0_ref, b0_ref, g0_ref, be0_ref),
            (W1_ref, b1_ref, g1_ref, be1_ref),
            (W2_ref, b2_ref, g2_ref, be2_ref)):
        x = lax.dot_general(x, W_ref[:], (((1,), (1,)), ((), ())),
                            preferred_element_type=jnp.float32) + b_ref[:]
        x = jnp.maximum(x, 0.0)
        mean = jnp.mean(x, axis=0, keepdims=True)
        var = jnp.mean((x - mean) ** 2, axis=0, keepdims=True)
        x = g_ref[:] * (x - mean) * lax.rsqrt(var + 1e-5) + be_ref[:]
    deep = lax.dot_general(x, Wo_ref[:], (((1,), (1,)), ((), ())),
                           preferred_element_type=jnp.float32)[:, 0]
    logit = fo_ref[:] + second + deep + bo_ref[0]
    out_ref[:] = 1.0 / (1.0 + jnp.exp(-logit))


def kernel(feature_ids, feature_values, first_order_table, second_order_table,
           W0, b0, gamma0, beta0, W1, b1, gamma1, beta1, W2, b2, gamma2, beta2,
           W_out, b_out):
    ids3 = feature_ids.reshape(B // L, L, F).transpose(0, 2, 1)
    idx = ids3.reshape(NW * NBLK * NCH, CH).astype(jnp.int32)
    vals = feature_values.reshape(B // L, L, F).transpose(0, 2, 1).reshape(-1)
    fo_flat = first_order_table.reshape(V)

    pooled, sumsq, fo = _sc_call(idx, vals, fo_flat, second_order_table)

    return pl.pallas_call(
        _tc_body,
        out_shape=jax.ShapeDtypeStruct((B,), jnp.float32),
    )(pooled, sumsq, fo, W0, b0, gamma0, beta0, W1, b1, gamma1, beta1,
      W2, b2, gamma2, beta2, W_out, b_out)


# speedup vs baseline: 1.0022x; 1.0022x over previous
import functools

import jax
import jax.numpy as jnp
from jax import lax
from jax.experimental import pallas as pl
from jax.experimental.pallas import tpu as pltpu
from jax.experimental.pallas import tpu_sc as plsc

B, F = 4096, 26
V, D = 100000, 64
L = 16
NC, NS = 2, 16
NW = NC * NS
ROWS_W = B // NW
NBLK = ROWS_W // L
IDX_BLK = F * L
NCH = 4
CH = IDX_BLK // NCH
DCH = D // L


def _sc_body(idx_hbm, vals_hbm, vsplat_hbm, fo_hbm, so_hbm,
             pooled_hbm, sumsq_hbm, fo_out_hbm,
             idx_v, vals_v, vsplat_v, emb_v, fo_v, pooled_v, sumsq_v, foacc_v,
             sem_e, sem_f):
    wid = lax.axis_index("s") * NC + lax.axis_index("c")
    pltpu.sync_copy(idx_hbm.at[pl.ds(wid * (NBLK * NCH), NBLK * NCH)], idx_v)
    pltpu.sync_copy(vals_hbm.at[pl.ds(wid * (NBLK * IDX_BLK), NBLK * IDX_BLK)],
                    vals_v)
    pltpu.sync_copy(
        vsplat_hbm.at[pl.ds(wid * (NBLK * IDX_BLK), NBLK * IDX_BLK)], vsplat_v)

    def block_body(blk, carry):
        for c in range(NCH):
            pltpu.async_copy(so_hbm.at[idx_v.at[blk * NCH + c]],
                             emb_v.at[pl.ds(c * CH, CH)], sem_e)
        for c in range(NCH):
            pltpu.async_copy(fo_hbm.at[idx_v.at[blk * NCH + c]],
                             fo_v.at[pl.ds(c * CH, CH)], sem_f)
        for c in range(NCH):
            pltpu.make_async_copy(so_hbm.at[idx_v.at[blk * NCH + c]],
                                  emb_v.at[pl.ds(c * CH, CH)], sem_e).wait()
        for c in range(NCH):
            pltpu.make_async_copy(fo_hbm.at[idx_v.at[blk * NCH + c]],
                                  fo_v.at[pl.ds(c * CH, CH)], sem_f).wait()

        vbase = blk * IDX_BLK
        acc = jnp.zeros((L,), jnp.float32)
        for f in range(F):
            acc = acc + fo_v[pl.ds(f * L, L)] * vals_v[pl.ds(vbase + f * L, L)]
        foacc_v[pl.ds(blk * L, L)] = acc

        def row_body(r, rc):
            row = blk * L + r
            accs = [jnp.zeros((L,), jnp.float32) for _ in range(2 * DCH)]
            for f in range(F):
                vv = vsplat_v[vbase + f * L + r, :]
                for c in range(DCH):
                    t = emb_v[f * L + r, pl.ds(c * L, L)] * vv
                    accs[c] = accs[c] + t
                    accs[DCH + c] = accs[DCH + c] + t * t
            for c in range(DCH):
                pooled_v[row, pl.ds(c * L, L)] = accs[c]
                sumsq_v[row, pl.ds(c * L, L)] = accs[DCH + c]
            return rc

        lax.fori_loop(0, L, row_body, 0)
        return carry

    lax.fori_loop(0, NBLK, block_body, 0)

    pltpu.sync_copy(pooled_v, pooled_hbm.at[pl.ds(wid * ROWS_W, ROWS_W)])
    pltpu.sync_copy(sumsq_v, sumsq_hbm.at[pl.ds(wid * ROWS_W, ROWS_W)])
    pltpu.sync_copy(foacc_v, fo_out_hbm.at[pl.ds(wid * ROWS_W, ROWS_W)])


@functools.cache
def _get_sc_call():
    return pl.kernel(
        _sc_body,
        out_type=(
            jax.ShapeDtypeStruct((B, D), jnp.float32),
            jax.ShapeDtypeStruct((B, D), jnp.float32),
            jax.ShapeDtypeStruct((B,), jnp.float32),
        ),
        mesh=plsc.VectorSubcoreMesh(core_axis_name="c", subcore_axis_name="s"),
        compiler_params=pltpu.CompilerParams(use_tc_tiling_on_sc=False),
        scratch_types=(
            pltpu.VMEM((NBLK * NCH, CH), jnp.int32),
            pltpu.VMEM((NBLK * IDX_BLK,), jnp.float32),
            pltpu.VMEM((NBLK * IDX_BLK, L), jnp.float32),
            pltpu.VMEM((IDX_BLK, D), jnp.float32),
            pltpu.VMEM((IDX_BLK,), jnp.float32),
            pltpu.VMEM((ROWS_W, D), jnp.float32),
            pltpu.VMEM((ROWS_W, D), jnp.float32),
            pltpu.VMEM((ROWS_W,), jnp.float32),
            pltpu.SemaphoreType.DMA,
            pltpu.SemaphoreType.DMA,
        ),
    )


def _tc_body(pooled_ref, sumsq_ref, fo_ref,
             W0_ref, b0_ref, g0_ref, be0_ref,
             W1_ref, b1_ref, g1_ref, be1_ref,
             W2_ref, b2_ref, g2_ref, be2_ref,
             Wo_ref, bo_ref, out_ref):
    p = pooled_ref[:]
    second = 0.5 * jnp.sum(p * p - sumsq_ref[:], axis=1)
    x = p
    for W_ref, b_ref, g_ref, be_ref in (
            (W0_ref, b0_ref, g0_ref, be0_ref),
            (W1_ref, b1_ref, g1_ref, be1_ref),
            (W2_ref, b2_ref, g2_ref, be2_ref)):
        x = lax.dot_general(x, W_ref[:], (((1,), (1,)), ((), ())),
                            preferred_element_type=jnp.float32) + b_ref[:]
        x = jnp.maximum(x, 0.0)
        mean = jnp.mean(x, axis=0, keepdims=True)
        var = jnp.mean((x - mean) ** 2, axis=0, keepdims=True)
        x = g_ref[:] * (x - mean) * lax.rsqrt(var + 1e-5) + be_ref[:]
    deep = lax.dot_general(x, Wo_ref[:], (((1,), (1,)), ((), ())),
                           preferred_element_type=jnp.float32)[:, 0]
    logit = fo_ref[:] + second + deep + bo_ref[0]
    out_ref[:] = 1.0 / (1.0 + jnp.exp(-logit))


def kernel(feature_ids, feature_values, first_order_table, second_order_table,
           W0, b0, gamma0, beta0, W1, b1, gamma1, beta1, W2, b2, gamma2, beta2,
           W_out, b_out):
    ids3 = feature_ids.reshape(B // L, L, F).transpose(0, 2, 1)
    idx = ids3.reshape(NW * NBLK * NCH, CH).astype(jnp.int32)
    vals = feature_values.reshape(B // L, L, F).transpose(0, 2, 1).reshape(-1)
    vsplat = jnp.broadcast_to(vals[:, None], (B * F, L))
    fo_flat = first_order_table.reshape(V)

    pooled, sumsq, fo = _get_sc_call()(idx, vals, vsplat, fo_flat,
                                       second_order_table)

    return pl.pallas_call(
        _tc_body,
        out_shape=jax.ShapeDtypeStruct((B,), jnp.float32),
    )(pooled, sumsq, fo, W0, b0, gamma0, beta0, W1, b1, gamma1, beta1,
      W2, b2, gamma2, beta2, W_out, b_out)
